# Initial kernel scaffold; baseline (speedup 1.0000x reference)
import jax
import jax.numpy as jnp
from jax.experimental import pallas as pl


def kernel(edge_index, emb_weight):
    raise NotImplementedError("write your pallas kernel here")


# speedup vs baseline: 8.3736x; 8.3736x over previous
import functools

import jax
import jax.numpy as jnp
from jax import lax
from jax.experimental import pallas as pl
from jax.experimental.pallas import tpu as pltpu
from jax.experimental.pallas import tpu_sc as plsc

N = 50000
D = 64
Q = 16
NQ = 4
E = 800000
NLAYERS = 3
NC, NS = 2, 16

GRP = 1024
EP = 819200
NGL = EP // GRP // NS
NGD = EP // GRP // (NC * NS)
ACC_N = 50176
TRASH = N
WB = 1000

_mesh = plsc.VectorSubcoreMesh(core_axis_name="c", subcore_axis_name="s")
_sc_params = pltpu.CompilerParams(use_tc_tiling_on_sc=False)


@functools.partial(
    pl.kernel,
    out_type=jax.ShapeDtypeStruct((NC, N, Q), jnp.float32),
    mesh=_mesh,
    compiler_params=_sc_params,
    scratch_types=[
        pltpu.VMEM((GRP, Q), jnp.float32),
        pltpu.VMEM((GRP,), jnp.int32),
        pltpu.VMEM_SHARED((ACC_N, Q), jnp.float32),
    ],
)
def _deg_kernel(dst_hbm, out_hbm, ones_v, idx_v, acc):
    c = lax.axis_index("c")
    s = lax.axis_index("s")
    w = c * NS + s

    @pl.loop(0, GRP)
    def _(i):
        ones_v[i] = jnp.zeros((Q,), jnp.float32)

    @pl.loop(s, ACC_N // GRP, step=NS)
    def _(k):
        pltpu.sync_copy(ones_v, acc.at[pl.ds(k * GRP, GRP)])

    @pl.loop(0, GRP)
    def _(i):
        ones_v[i] = jnp.ones((Q,), jnp.float32)

    plsc.subcore_barrier()

    @pl.loop(0, NGD)
    def _(k):
        base_e = (w + (NC * NS) * k) * GRP
        pltpu.sync_copy(dst_hbm.at[pl.ds(base_e, GRP)], idx_v)
        pltpu.sync_copy(ones_v, acc.at[idx_v], add=True)

    plsc.subcore_barrier()

    @pl.loop(s, N // WB, step=NS)
    def _(k):
        pltpu.sync_copy(
            acc.at[pl.ds(k * WB, WB)],
            out_hbm.at[c, pl.ds(k * WB, WB)],
        )


@functools.partial(
    pl.kernel,
    out_type=jax.ShapeDtypeStruct((NQ, N, Q), jnp.float32),
    mesh=_mesh,
    compiler_params=_sc_params,
    scratch_types=[
        pltpu.VMEM((GRP, Q), jnp.float32),
        pltpu.VMEM((GRP,), jnp.int32),
        pltpu.VMEM((GRP,), jnp.int32),
        pltpu.VMEM_SHARED((ACC_N, Q), jnp.float32),
    ],
)
def _layer_kernel(u_hbm, src_hbm, dst_hbm, out_hbm,
                  rows_v, si_v, di_v, acc):
    c = lax.axis_index("c")
    s = lax.axis_index("s")

    for q in range(2):
        g = 2 * c + q

        @pl.loop(0, GRP)
        def _(i):
            rows_v[i] = jnp.zeros((Q,), jnp.float32)

        @pl.loop(s, ACC_N // GRP, step=NS)
        def _(k):
            pltpu.sync_copy(rows_v, acc.at[pl.ds(k * GRP, GRP)])

        plsc.subcore_barrier()

        @pl.loop(0, NGL)
        def _(i):
            base_e = (s * NGL + i) * GRP
            pltpu.sync_copy(src_hbm.at[pl.ds(base_e, GRP)], si_v)
            pltpu.sync_copy(dst_hbm.at[pl.ds(base_e, GRP)], di_v)
            pltpu.sync_copy(u_hbm.at[g].at[si_v], rows_v)
            pltpu.sync_copy(rows_v, acc.at[di_v], add=True)

        plsc.subcore_barrier()

        @pl.loop(s, N // WB, step=NS)
        def _(k):
            pltpu.sync_copy(
                acc.at[pl.ds(k * WB, WB)],
                out_hbm.at[g, pl.ds(k * WB, WB)],
            )

        if q == 0:
            plsc.subcore_barrier()


_B = 2000


def _prep_tc(degp, emb):
    def body(degp_ref, emb_ref, dis_ref, dis2_ref, u_ref):
        dp = degp_ref[...]
        deg = dp[0, :, 0:1] + dp[1, :, 0:1]
        good = deg > 0
        d = jnp.where(good, lax.rsqrt(jnp.where(good, deg, 1.0)), 0.0)
        dis_ref[...] = d
        dis2_ref[...] = d * d
        e = emb_ref[...]
        for q in range(NQ):
            u_ref[q] = e[:, q * Q:(q + 1) * Q] * d

    return pl.pallas_call(
        body,
        grid=(N // _B,),
        in_specs=[
            pl.BlockSpec((2, _B, Q), lambda i: (0, i, 0)),
            pl.BlockSpec((_B, D), lambda i: (i, 0)),
        ],
        out_specs=[
            pl.BlockSpec((_B, 1), lambda i: (i, 0)),
            pl.BlockSpec((_B, 1), lambda i: (i, 0)),
            pl.BlockSpec((NQ, _B, Q), lambda i: (0, i, 0)),
        ],
        out_shape=[
            jax.ShapeDtypeStruct((N, 1), jnp.float32),
            jax.ShapeDtypeStruct((N, 1), jnp.float32),
            jax.ShapeDtypeStruct((NQ, N, Q), jnp.float32),
        ],
    )(degp, emb)


def _scale_tc(t, dis2):
    def body(t_ref, dis2_ref, u_ref):
        d2 = dis2_ref[...]
        for q in range(NQ):
            u_ref[q] = t_ref[q] * d2

    return pl.pallas_call(
        body,
        grid=(N // _B,),
        in_specs=[
            pl.BlockSpec((NQ, _B, Q), lambda i: (0, i, 0)),
            pl.BlockSpec((_B, 1), lambda i: (i, 0)),
        ],
        out_specs=pl.BlockSpec((NQ, _B, Q), lambda i: (0, i, 0)),
        out_shape=jax.ShapeDtypeStruct((NQ, N, Q), jnp.float32),
    )(t, dis2)


def _final_tc(emb, t1, t2, t3, dis):
    def body(emb_ref, t1_ref, t2_ref, t3_ref, dis_ref, out_ref):
        d = dis_ref[...]
        e = emb_ref[...]
        total = jnp.concatenate(
            [t1_ref[q] + t2_ref[q] + t3_ref[q] for q in range(NQ)], axis=1)
        out_ref[...] = (e + total * d) * 0.25

    spec_t = pl.BlockSpec((NQ, _B, Q), lambda i: (0, i, 0))
    return pl.pallas_call(
        body,
        grid=(N // _B,),
        in_specs=[
            pl.BlockSpec((_B, D), lambda i: (i, 0)),
            spec_t, spec_t, spec_t,
            pl.BlockSpec((_B, 1), lambda i: (i, 0)),
        ],
        out_specs=pl.BlockSpec((_B, D), lambda i: (i, 0)),
        out_shape=jax.ShapeDtypeStruct((N, D), jnp.float32),
    )(emb, t1, t2, t3, dis)


def kernel(edge_index, emb_weight):
    pad = EP - E
    src_p = jnp.concatenate([edge_index[0], jnp.zeros((pad,), jnp.int32)])
    dst_p = jnp.concatenate(
        [edge_index[1], jnp.full((pad,), TRASH, jnp.int32)])

    degp = _deg_kernel(dst_p)
    dis, dis2, u = _prep_tc(degp, emb_weight)

    ts = []
    for l in range(NLAYERS):
        t = _layer_kernel(u, src_p, dst_p)
        ts.append(t)
        if l < NLAYERS - 1:
            u = _scale_tc(t, dis2)

    out = _final_tc(emb_weight, ts[0], ts[1], ts[2], dis)
    return (emb_weight, out)


# speedup vs baseline: 8.4125x; 1.0046x over previous
import functools

import jax
import jax.numpy as jnp
from jax import lax
from jax.experimental import pallas as pl
from jax.experimental.pallas import tpu as pltpu
from jax.experimental.pallas import tpu_sc as plsc

N = 50000
D = 64
Q = 16
NQ = 4
E = 800000
NLAYERS = 3
NC, NS = 2, 16

GRP = 1024
EP = 819200
NGL = EP // GRP // NS
NGD = EP // GRP // (NC * NS)
ACC_N = 50176
TRASH = N
WB = 1000

_mesh = plsc.VectorSubcoreMesh(core_axis_name="c", subcore_axis_name="s")
_sc_params = pltpu.CompilerParams(use_tc_tiling_on_sc=False)


@functools.partial(
    pl.kernel,
    out_type=jax.ShapeDtypeStruct((NC, N, Q), jnp.float32),
    mesh=_mesh,
    compiler_params=_sc_params,
    scratch_types=[
        pltpu.VMEM((GRP, Q), jnp.float32),
        pltpu.VMEM((GRP,), jnp.int32),
        pltpu.VMEM_SHARED((ACC_N, Q), jnp.float32),
    ],
)
def _deg_kernel(dst_hbm, out_hbm, ones_v, idx_v, acc):
    c = lax.axis_index("c")
    s = lax.axis_index("s")
    w = c * NS + s

    @pl.loop(0, GRP)
    def _(i):
        ones_v[i] = jnp.zeros((Q,), jnp.float32)

    @pl.loop(s, ACC_N // GRP, step=NS)
    def _(k):
        pltpu.sync_copy(ones_v, acc.at[pl.ds(k * GRP, GRP)])

    @pl.loop(0, GRP)
    def _(i):
        ones_v[i] = jnp.ones((Q,), jnp.float32)

    plsc.subcore_barrier()

    @pl.loop(0, NGD)
    def _(k):
        base_e = (w + (NC * NS) * k) * GRP
        pltpu.sync_copy(dst_hbm.at[pl.ds(base_e, GRP)], idx_v)
        pltpu.sync_copy(ones_v, acc.at[idx_v], add=True)

    plsc.subcore_barrier()

    @pl.loop(s, N // WB, step=NS)
    def _(k):
        pltpu.sync_copy(
            acc.at[pl.ds(k * WB, WB)],
            out_hbm.at[c, pl.ds(k * WB, WB)],
        )


@functools.partial(
    pl.kernel,
    out_type=jax.ShapeDtypeStruct((NQ, N, Q), jnp.float32),
    mesh=_mesh,
    compiler_params=_sc_params,
    scratch_types=[
        pltpu.VMEM((GRP, Q), jnp.float32),
        pltpu.VMEM((GRP, Q), jnp.float32),
        pltpu.VMEM((GRP,), jnp.int32),
        pltpu.VMEM((GRP,), jnp.int32),
        pltpu.VMEM((GRP,), jnp.int32),
        pltpu.VMEM((GRP,), jnp.int32),
        pltpu.SemaphoreType.DMA,
        pltpu.SemaphoreType.DMA,
        pltpu.VMEM_SHARED((ACC_N, Q), jnp.float32),
    ],
)
def _layer_kernel(u_hbm, src_hbm, dst_hbm, out_hbm,
                  rows0, rows1, si0, si1, di0, di1, sem0, sem1, acc):
    c = lax.axis_index("c")
    s = lax.axis_index("s")
    bufs = ((rows0, si0, di0, sem0), (rows1, si1, di1, sem1))

    for q in range(2):
        g = 2 * c + q

        @pl.loop(0, GRP)
        def _(i):
            rows0[i] = jnp.zeros((Q,), jnp.float32)
            rows1[i] = jnp.zeros((Q,), jnp.float32)

        @pl.loop(0, GRP // 16)
        def _(i):
            di0[pl.ds(i * 16, 16)] = jnp.zeros((16,), jnp.int32)
            di1[pl.ds(i * 16, 16)] = jnp.zeros((16,), jnp.int32)

        @pl.loop(s, ACC_N // GRP, step=NS)
        def _(k):
            pltpu.sync_copy(rows0, acc.at[pl.ds(k * GRP, GRP)])

        plsc.subcore_barrier()

        pltpu.async_copy(rows0, acc.at[di0], sem0, add=True)
        pltpu.async_copy(rows1, acc.at[di1], sem1, add=True)

        @pl.loop(0, NGL, step=2)
        def _(i):
            for b, (rows, si, di, sem) in enumerate(bufs):
                base_e = (s * NGL + i + b) * GRP
                pltpu.make_async_copy(rows, acc.at[di], sem).wait()
                pltpu.sync_copy(src_hbm.at[pl.ds(base_e, GRP)], si)
                pltpu.sync_copy(dst_hbm.at[pl.ds(base_e, GRP)], di)
                pltpu.sync_copy(u_hbm.at[g].at[si], rows)
                pltpu.async_copy(rows, acc.at[di], sem, add=True)

        pltpu.make_async_copy(rows0, acc.at[di0], sem0).wait()
        pltpu.make_async_copy(rows1, acc.at[di1], sem1).wait()

        plsc.subcore_barrier()

        @pl.loop(s, N // WB, step=NS)
        def _(k):
            pltpu.sync_copy(
                acc.at[pl.ds(k * WB, WB)],
                out_hbm.at[g, pl.ds(k * WB, WB)],
            )

        if q == 0:
            plsc.subcore_barrier()


_B = 2000


def _prep_tc(degp, emb):
    def body(degp_ref, emb_ref, dis_ref, dis2_ref, u_ref):
        dp = degp_ref[...]
        deg = dp[0, :, 0:1] + dp[1, :, 0:1]
        good = deg > 0
        d = jnp.where(good, lax.rsqrt(jnp.where(good, deg, 1.0)), 0.0)
        dis_ref[...] = d
        dis2_ref[...] = d * d
        e = emb_ref[...]
        for q in range(NQ):
            u_ref[q] = e[:, q * Q:(q + 1) * Q] * d

    return pl.pallas_call(
        body,
        grid=(N // _B,),
        in_specs=[
            pl.BlockSpec((2, _B, Q), lambda i: (0, i, 0)),
            pl.BlockSpec((_B, D), lambda i: (i, 0)),
        ],
        out_specs=[
            pl.BlockSpec((_B, 1), lambda i: (i, 0)),
            pl.BlockSpec((_B, 1), lambda i: (i, 0)),
            pl.BlockSpec((NQ, _B, Q), lambda i: (0, i, 0)),
        ],
        out_shape=[
            jax.ShapeDtypeStruct((N, 1), jnp.float32),
            jax.ShapeDtypeStruct((N, 1), jnp.float32),
            jax.ShapeDtypeStruct((NQ, N, Q), jnp.float32),
        ],
    )(degp, emb)


def _scale_tc(t, dis2):
    def body(t_ref, dis2_ref, u_ref):
        d2 = dis2_ref[...]
        for q in range(NQ):
            u_ref[q] = t_ref[q] * d2

    return pl.pallas_call(
        body,
        grid=(N // _B,),
        in_specs=[
            pl.BlockSpec((NQ, _B, Q), lambda i: (0, i, 0)),
            pl.BlockSpec((_B, 1), lambda i: (i, 0)),
        ],
        out_specs=pl.BlockSpec((NQ, _B, Q), lambda i: (0, i, 0)),
        out_shape=jax.ShapeDtypeStruct((NQ, N, Q), jnp.float32),
    )(t, dis2)


def _final_tc(emb, t1, t2, t3, dis):
    def body(emb_ref, t1_ref, t2_ref, t3_ref, dis_ref, out_ref):
        d = dis_ref[...]
        e = emb_ref[...]
        total = jnp.concatenate(
            [t1_ref[q] + t2_ref[q] + t3_ref[q] for q in range(NQ)], axis=1)
        out_ref[...] = (e + total * d) * 0.25

    spec_t = pl.BlockSpec((NQ, _B, Q), lambda i: (0, i, 0))
    return pl.pallas_call(
        body,
        grid=(N // _B,),
        in_specs=[
            pl.BlockSpec((_B, D), lambda i: (i, 0)),
            spec_t, spec_t, spec_t,
            pl.BlockSpec((_B, 1), lambda i: (i, 0)),
        ],
        out_specs=pl.BlockSpec((_B, D), lambda i: (i, 0)),
        out_shape=jax.ShapeDtypeStruct((N, D), jnp.float32),
    )(emb, t1, t2, t3, dis)


def kernel(edge_index, emb_weight):
    pad = EP - E
    src_p = jnp.concatenate([edge_index[0], jnp.zeros((pad,), jnp.int32)])
    dst_p = jnp.concatenate(
        [edge_index[1], jnp.full((pad,), TRASH, jnp.int32)])

    degp = _deg_kernel(dst_p)
    dis, dis2, u = _prep_tc(degp, emb_weight)

    ts = []
    for l in range(NLAYERS):
        t = _layer_kernel(u, src_p, dst_p)
        ts.append(t)
        if l < NLAYERS - 1:
            u = _scale_tc(t, dis2)

    out = _final_tc(emb_weight, ts[0], ts[1], ts[2], dis)
    return (emb_weight, out)


# speedup vs baseline: 9.1626x; 1.0892x over previous
import functools

import jax
import jax.numpy as jnp
from jax import lax
from jax.experimental import pallas as pl
from jax.experimental.pallas import tpu as pltpu
from jax.experimental.pallas import tpu_sc as plsc

N = 50000
D = 64
Q = 16
NQ = 4
E = 800000
NLAYERS = 3
NC, NS = 2, 16

GRP = 1024
EP = 819200
NGL = EP // GRP // NS
NGD = EP // GRP // (NC * NS)
ACC_N = 50176
TRASH = N
WB = 1000

_mesh = plsc.VectorSubcoreMesh(core_axis_name="c", subcore_axis_name="s")
_sc_params = pltpu.CompilerParams(use_tc_tiling_on_sc=False)


@functools.partial(
    pl.kernel,
    out_type=jax.ShapeDtypeStruct((NC, N, Q), jnp.float32),
    mesh=_mesh,
    compiler_params=_sc_params,
    scratch_types=[
        pltpu.VMEM((GRP, Q), jnp.float32),
        pltpu.VMEM((GRP,), jnp.int32),
        pltpu.VMEM_SHARED((ACC_N, Q), jnp.float32),
    ],
)
def _deg_kernel(dst_hbm, out_hbm, ones_v, idx_v, acc):
    c = lax.axis_index("c")
    s = lax.axis_index("s")
    w = c * NS + s

    @pl.loop(0, GRP)
    def _(i):
        ones_v[i] = jnp.zeros((Q,), jnp.float32)

    @pl.loop(s, ACC_N // GRP, step=NS)
    def _(k):
        pltpu.sync_copy(ones_v, acc.at[pl.ds(k * GRP, GRP)])

    @pl.loop(0, GRP)
    def _(i):
        ones_v[i] = jnp.ones((Q,), jnp.float32)

    plsc.subcore_barrier()

    @pl.loop(0, NGD)
    def _(k):
        base_e = (w + (NC * NS) * k) * GRP
        pltpu.sync_copy(dst_hbm.at[pl.ds(base_e, GRP)], idx_v)
        pltpu.sync_copy(ones_v, acc.at[idx_v], add=True)

    plsc.subcore_barrier()

    @pl.loop(s, N // WB, step=NS)
    def _(k):
        pltpu.sync_copy(
            acc.at[pl.ds(k * WB, WB)],
            out_hbm.at[c, pl.ds(k * WB, WB)],
        )


@functools.partial(
    pl.kernel,
    out_type=jax.ShapeDtypeStruct((NQ, N, Q), jnp.float32),
    mesh=_mesh,
    compiler_params=_sc_params,
    scratch_types=[
        pltpu.VMEM((GRP, Q), jnp.float32),
        pltpu.VMEM((GRP, Q), jnp.float32),
        pltpu.VMEM((GRP,), jnp.int32),
        pltpu.VMEM((GRP,), jnp.int32),
        pltpu.VMEM((GRP,), jnp.int32),
        pltpu.VMEM((GRP,), jnp.int32),
        pltpu.SemaphoreType.DMA,
        pltpu.SemaphoreType.DMA,
        pltpu.SemaphoreType.DMA,
        pltpu.SemaphoreType.DMA,
        pltpu.VMEM_SHARED((ACC_N, Q), jnp.float32),
    ],
)
def _layer_kernel(u_hbm, src_hbm, dst_hbm, out_hbm,
                  rows0, rows1, si0, si1, di0, di1, sem0, sem1,
                  semg0, semg1, acc):
    c = lax.axis_index("c")
    s = lax.axis_index("s")
    bufs = ((rows0, si0, di0, sem0, semg0), (rows1, si1, di1, sem1, semg1))

    for q in range(2):
        g = 2 * c + q

        @pl.loop(0, GRP)
        def _(i):
            rows0[i] = jnp.zeros((Q,), jnp.float32)
            rows1[i] = jnp.zeros((Q,), jnp.float32)

        @pl.loop(0, GRP // 16)
        def _(i):
            di0[pl.ds(i * 16, 16)] = jnp.zeros((16,), jnp.int32)
            di1[pl.ds(i * 16, 16)] = jnp.zeros((16,), jnp.int32)

        @pl.loop(s, ACC_N // GRP, step=NS)
        def _(k):
            pltpu.sync_copy(rows0, acc.at[pl.ds(k * GRP, GRP)])

        plsc.subcore_barrier()

        pltpu.async_copy(rows0, acc.at[di0], sem0, add=True)
        pltpu.async_copy(rows1, acc.at[di1], sem1, add=True)

        @pl.loop(0, NGL, step=2)
        def _(i):
            for b, (rows, si, di, sem, semg) in enumerate(bufs):
                base_e = (s * NGL + i + b) * GRP
                pltpu.make_async_copy(rows, acc.at[di], sem).wait()
                pltpu.sync_copy(src_hbm.at[pl.ds(base_e, GRP)], si)
                pltpu.sync_copy(dst_hbm.at[pl.ds(base_e, GRP)], di)
                pltpu.async_copy(u_hbm.at[g].at[si], rows, semg)
            for b, (rows, si, di, sem, semg) in enumerate(bufs):
                pltpu.make_async_copy(u_hbm.at[g].at[si], rows, semg).wait()
                pltpu.async_copy(rows, acc.at[di], sem, add=True)

        pltpu.make_async_copy(rows0, acc.at[di0], sem0).wait()
        pltpu.make_async_copy(rows1, acc.at[di1], sem1).wait()

        plsc.subcore_barrier()

        @pl.loop(s, N // WB, step=NS)
        def _(k):
            pltpu.sync_copy(
                acc.at[pl.ds(k * WB, WB)],
                out_hbm.at[g, pl.ds(k * WB, WB)],
            )

        if q == 0:
            plsc.subcore_barrier()


_B = 2000


def _prep_tc(degp, emb):
    def body(degp_ref, emb_ref, dis_ref, dis2_ref, u_ref):
        dp = degp_ref[...]
        deg = dp[0, :, 0:1] + dp[1, :, 0:1]
        good = deg > 0
        d = jnp.where(good, lax.rsqrt(jnp.where(good, deg, 1.0)), 0.0)
        dis_ref[...] = d
        dis2_ref[...] = d * d
        e = emb_ref[...]
        for q in range(NQ):
            u_ref[q] = e[:, q * Q:(q + 1) * Q] * d

    return pl.pallas_call(
        body,
        grid=(N // _B,),
        in_specs=[
            pl.BlockSpec((2, _B, Q), lambda i: (0, i, 0)),
            pl.BlockSpec((_B, D), lambda i: (i, 0)),
        ],
        out_specs=[
            pl.BlockSpec((_B, 1), lambda i: (i, 0)),
            pl.BlockSpec((_B, 1), lambda i: (i, 0)),
            pl.BlockSpec((NQ, _B, Q), lambda i: (0, i, 0)),
        ],
        out_shape=[
            jax.ShapeDtypeStruct((N, 1), jnp.float32),
            jax.ShapeDtypeStruct((N, 1), jnp.float32),
            jax.ShapeDtypeStruct((NQ, N, Q), jnp.float32),
        ],
    )(degp, emb)


def _scale_tc(t, dis2):
    def body(t_ref, dis2_ref, u_ref):
        d2 = dis2_ref[...]
        for q in range(NQ):
            u_ref[q] = t_ref[q] * d2

    return pl.pallas_call(
        body,
        grid=(N // _B,),
        in_specs=[
            pl.BlockSpec((NQ, _B, Q), lambda i: (0, i, 0)),
            pl.BlockSpec((_B, 1), lambda i: (i, 0)),
        ],
        out_specs=pl.BlockSpec((NQ, _B, Q), lambda i: (0, i, 0)),
        out_shape=jax.ShapeDtypeStruct((NQ, N, Q), jnp.float32),
    )(t, dis2)


def _final_tc(emb, t1, t2, t3, dis):
    def body(emb_ref, t1_ref, t2_ref, t3_ref, dis_ref, out_ref):
        d = dis_ref[...]
        e = emb_ref[...]
        total = jnp.concatenate(
            [t1_ref[q] + t2_ref[q] + t3_ref[q] for q in range(NQ)], axis=1)
        out_ref[...] = (e + total * d) * 0.25

    spec_t = pl.BlockSpec((NQ, _B, Q), lambda i: (0, i, 0))
    return pl.pallas_call(
        body,
        grid=(N // _B,),
        in_specs=[
            pl.BlockSpec((_B, D), lambda i: (i, 0)),
            spec_t, spec_t, spec_t,
            pl.BlockSpec((_B, 1), lambda i: (i, 0)),
        ],
        out_specs=pl.BlockSpec((_B, D), lambda i: (i, 0)),
        out_shape=jax.ShapeDtypeStruct((N, D), jnp.float32),
    )(emb, t1, t2, t3, dis)


def kernel(edge_index, emb_weight):
    pad = EP - E
    src_p = jnp.concatenate([edge_index[0], jnp.zeros((pad,), jnp.int32)])
    dst_p = jnp.concatenate(
        [edge_index[1], jnp.full((pad,), TRASH, jnp.int32)])

    degp = _deg_kernel(dst_p)
    dis, dis2, u = _prep_tc(degp, emb_weight)

    ts = []
    for l in range(NLAYERS):
        t = _layer_kernel(u, src_p, dst_p)
        ts.append(t)
        if l < NLAYERS - 1:
            u = _scale_tc(t, dis2)

    out = _final_tc(emb_weight, ts[0], ts[1], ts[2], dis)
    return (emb_weight, out)
